# Initial kernel scaffold; baseline (speedup 1.0000x reference)
import jax
import jax.numpy as jnp
from jax.experimental import pallas as pl


def kernel(x_t, active_sites, edge_index, edge_attr, conds, time_node, params):
    raise NotImplementedError("write your pallas kernel here")


# speedup vs baseline: 9.4369x; 9.4369x over previous
import functools
import math

import jax
import jax.numpy as jnp
from jax import lax
from jax.experimental import pallas as pl
from jax.experimental.pallas import tpu as pltpu
from jax.experimental.pallas import tpu_sc as plsc

N_NODES = 50000
N_PAD = 50176
N_EDGES = 3200000
NC, NS, LANES = 2, 16, 16
NW = NC * NS
EDGES_PER_W = N_EDGES // NW
CHUNK = 1000
NCHUNK = EDGES_PER_W // CHUNK
ROWS_PER_SUB = N_PAD // NS
TEMB_SCALE = math.pi / 1000.0
BE = 3200
BN = 2000
F32 = jnp.float32


def _sc_edge_body(src, dst, pre16, xw16, z16, out,
                  src_v, dst_v, gat_v, table, acc, sem):
    cid = lax.axis_index("c")
    sid = lax.axis_index("s")
    w = sid * NC + cid
    r0 = pl.multiple_of(sid * ROWS_PER_SUB, 8)
    pltpu.sync_copy(xw16.at[pl.ds(r0, ROWS_PER_SUB)],
                    table.at[pl.ds(r0, ROWS_PER_SUB)])
    pltpu.sync_copy(z16.at[pl.ds(r0, ROWS_PER_SUB)],
                    acc.at[pl.ds(r0, ROWS_PER_SUB)])
    plsc.subcore_barrier()

    zero16 = jnp.zeros((LANES,), F32)

    def chunk_body(k, _):
        base = pl.multiple_of(w * EDGES_PER_W + k * CHUNK, 8)
        pltpu.sync_copy(src.at[pl.ds(base, CHUNK)], src_v)
        pltpu.sync_copy(dst.at[pl.ds(base, CHUNK)], dst_v)
        pltpu.sync_copy(pre16.at[pl.ds(base, CHUNK)], gat_v)
        pltpu.async_copy(table.at[src_v], gat_v, sem, add=True).wait()

        def edge_body(i, _):
            gat_v[i, :] = jnp.maximum(gat_v[i, :], zero16)
            return 0

        lax.fori_loop(0, CHUNK, edge_body, 0, unroll=8)
        pltpu.sync_copy(gat_v, acc.at[dst_v], add=True)
        return 0

    lax.fori_loop(0, NCHUNK, chunk_body, 0)
    plsc.subcore_barrier()
    pltpu.sync_copy(acc.at[pl.ds(r0, ROWS_PER_SUB)],
                    out.at[cid, pl.ds(r0, ROWS_PER_SUB)])


def _sc_edge(src, dst, pre16, xw16, z16):
    fn = pl.kernel(
        _sc_edge_body,
        out_type=jax.ShapeDtypeStruct((2, N_PAD, 16), F32),
        mesh=plsc.VectorSubcoreMesh(core_axis_name="c", subcore_axis_name="s",
                                    num_cores=NC, num_subcores=NS),
        compiler_params=pltpu.CompilerParams(use_tc_tiling_on_sc=False),
        scratch_types=[
            pltpu.VMEM((CHUNK,), jnp.int32),
            pltpu.VMEM((CHUNK,), jnp.int32),
            pltpu.VMEM((CHUNK, 16), F32),
            pltpu.VMEM_SHARED((N_PAD, 16), F32),
            pltpu.VMEM_SHARED((N_PAD, 16), F32),
            pltpu.SemaphoreType.DMA,
        ],
    )
    return fn(src, dst, pre16, xw16, z16)


def _edge_prep_body(ea_ref, w_ref, b_ref, o0, o1, o2):
    h = jnp.dot(ea_ref[...], w_ref[...],
                preferred_element_type=F32) + b_ref[...]
    one = jnp.full((BE, 1), 1.0, F32)
    zer = jnp.zeros((BE, 7), F32)
    o0[...] = jnp.concatenate([h[:, 0:8], one, zer], axis=1)
    o1[...] = jnp.concatenate([h[:, 8:16], one * 0.0, zer], axis=1)
    o2[...] = jnp.concatenate([h[:, 16:24], one * 0.0, zer], axis=1)


def _edge_prep(ea, wcat, bcat):
    return pl.pallas_call(
        _edge_prep_body,
        grid=(N_EDGES // BE,),
        in_specs=[pl.BlockSpec((BE, 7), lambda i: (i, 0)),
                  pl.BlockSpec((7, 24), lambda i: (0, 0)),
                  pl.BlockSpec((1, 24), lambda i: (0, 0))],
        out_specs=[pl.BlockSpec((BE, 16), lambda i: (i, 0))] * 3,
        out_shape=[jax.ShapeDtypeStruct((N_EDGES, 16), F32)] * 3,
    )(ea, wcat, bcat)


def _xw0_body(x_ref, w_ref, o_ref):
    o_ref[...] = jnp.dot(x_ref[...], w_ref[...], preferred_element_type=F32)


def _xw0(x0, w0p):
    return pl.pallas_call(
        _xw0_body,
        grid=(N_NODES // BN,),
        in_specs=[pl.BlockSpec((BN, 7), lambda i: (i, 0)),
                  pl.BlockSpec((7, 16), lambda i: (0, 0))],
        out_specs=pl.BlockSpec((BN, 16), lambda i: (i, 0)),
        out_shape=jax.ShapeDtypeStruct((N_PAD, 16), F32),
    )(x0, w0p)


def _node_body(has_next, sp0, sp1, c0, c1, x, tn, cond,
               w2psi, b2psi, p1x, p1a, p1t0, p1t1, p1b, p2, p2b,
               wc, bc, g1, g1b, g2, g2b, e1, e1b, e2, e2b,
               *rest):
    if has_next:
        wnext, out_ref, xwn_ref = rest
    else:
        (out_ref,) = rest
    relu = lambda v: jnp.maximum(v, 0.0)
    dot = functools.partial(jnp.dot, preferred_element_type=F32)

    S = sp0[0, :, 0:8] + sp1[0, :, 0:8]
    cnt = c0[0, :, 8:9] + c1[0, :, 8:9]
    inv = 1.0 / jnp.maximum(cnt, 1.0)
    agg = (dot(S, w2psi[...]) + cnt * b2psi[...]) * inv
    ce = dot(cond[...], wc[...]) + bc[...]
    gam = dot(relu(dot(ce, g1[...]) + g1b[...]), g2[...]) + g2b[...]
    bet = dot(relu(dot(ce, e1[...]) + e1b[...]), e2[...]) + e2b[...]
    ang = tn[...] * TEMB_SCALE
    u = (dot(x[...], p1x[...]) + dot(agg, p1a[...])
         + jnp.cos(ang) * p1t0[...] + jnp.sin(ang) * p1t1[...] + p1b[...])
    h = dot(relu(u), p2[...]) + p2b[...]
    o = gam * h + bet
    out_ref[...] = o
    if has_next:
        xwn_ref[...] = dot(o, wnext[...])


def _node(l, Sp, S0, x, tn, conds, wl, out_dim, has_next):
    full = lambda a: pl.BlockSpec(a.shape, lambda i: (0,) * a.ndim)
    row = lambda w: pl.BlockSpec((BN, w), lambda i: (i, 0))
    p0 = pl.BlockSpec((1, BN, 16), lambda i: (0, i, 0))
    p1 = pl.BlockSpec((1, BN, 16), lambda i: (1, i, 0))
    weights = [wl['w2psi'], wl['b2psi'], wl['p1x'], wl['p1a'], wl['p1t0'],
               wl['p1t1'], wl['p1b'], wl['p2'], wl['p2b'], wl['wc'],
               wl['bc'], wl['g1'], wl['g1b'], wl['g2'], wl['g2b'],
               wl['e1'], wl['e1b'], wl['e2'], wl['e2b']]
    if has_next:
        weights.append(wl['wnext'])
    in_specs = ([p0, p1, p0, p1, row(x.shape[1]), row(1), row(4)]
                + [full(w) for w in weights])
    out_specs = [row(out_dim)]
    out_shape = [jax.ShapeDtypeStruct((N_NODES, out_dim), F32)]
    if has_next:
        out_specs.append(row(16))
        out_shape.append(jax.ShapeDtypeStruct((N_PAD, 16), F32))
    res = pl.pallas_call(
        functools.partial(_node_body, has_next),
        grid=(N_NODES // BN,),
        in_specs=in_specs,
        out_specs=out_specs,
        out_shape=out_shape,
    )(Sp, Sp, S0, S0, x, tn, conds, *weights)
    return res if has_next else (res[0], None)


_INS = (7, 8, 8)
_OUTS = (8, 8, 5)


def _prep_layer(p, in_dim, nxt_w1x):
    (w1, b1), (w2, b2) = p['psi']
    (q1, q1b), (q2, q2b) = p['phi']
    (g1, g1b), (g2, g2b) = p['gamma']
    (e1, e1b), (e2, e2b) = p['beta']
    wl = {
        'w1e': w1[in_dim:], 'b1': b1.reshape(1, -1),
        'w2psi': w2, 'b2psi': b2.reshape(1, -1),
        'p1x': q1[:in_dim], 'p1a': q1[in_dim:in_dim + 8],
        'p1t0': q1[in_dim + 8:in_dim + 9], 'p1t1': q1[in_dim + 9:in_dim + 10],
        'p1b': q1b.reshape(1, -1), 'p2': q2, 'p2b': q2b.reshape(1, -1),
        'g1': g1, 'g1b': g1b.reshape(1, -1), 'g2': g2, 'g2b': g2b.reshape(1, -1),
        'e1': e1, 'e1b': e1b.reshape(1, -1), 'e2': e2, 'e2b': e2b.reshape(1, -1),
    }
    if nxt_w1x is not None:
        wl['wnext'] = jnp.concatenate(
            [nxt_w1x, jnp.zeros_like(nxt_w1x)], axis=1)
    return wl


def kernel(x_t, active_sites, edge_index, edge_attr, conds, time_node, params):
    layers = [params['l0'], params['l1'], params['l2']]
    w1xs = [p['psi'][0][0][:din] for p, din in zip(layers, _INS)]
    wls = [_prep_layer(p, din, w1xs[i + 1] if i < 2 else None)
           for i, (p, din) in enumerate(zip(layers, _INS))]
    for wl in wls:
        wl['wc'] = params['cond'][0]
        wl['bc'] = params['cond'][1].reshape(1, -1)
    wcat = jnp.concatenate([wl['w1e'] for wl in wls], axis=1)
    bcat = jnp.concatenate([wl['b1'] for wl in wls], axis=1)

    x0 = jnp.concatenate([x_t, active_sites], axis=1)
    tn = time_node.reshape(N_NODES, 1)
    z16 = jnp.zeros((N_PAD, 16), F32)
    w0p = jnp.concatenate([w1xs[0], jnp.zeros((_INS[0], 8), F32)], axis=1)
    src = edge_index[0]
    dst = edge_index[1]

    pres = _edge_prep(edge_attr, wcat, bcat)
    xw16 = _xw0(x0, w0p)

    x = x0
    S0 = None
    for l in range(3):
        Sp = _sc_edge(src, dst, pres[l], xw16, z16)
        if l == 0:
            S0 = Sp
        x, xw16 = _node(l, Sp, S0, x, tn, conds, wls[l], _OUTS[l], l < 2)
    return x
